# baseline (speedup 1.0000x reference)
import jax
import jax.numpy as jnp
from jax.experimental import pallas as pl

_IM = 64
_N = _IM * _IM
_Q = 128
_M = 16
_ITERS = 32
_OFFS = tuple((di, dj) for di in (-1, 0, 1) for dj in (-1, 0, 1))


def _scene_kernel(xf, wc1, bc1, wc2, bc2, wk1q1, wk2q2, wk3q3, bkq3,
                  s0, out_ref):
    f32 = jnp.float32
    p = jax.lax.broadcasted_iota(jnp.int32, (_N, 1), 0)
    i = p // _IM
    j = p - i * _IM
    masks = []
    for (di, dj) in _OFFS:
        ii = i + di
        jj = j + dj
        ok = (ii >= 0) & (ii < _IM) & (jj >= 0) & (jj < _IM)
        masks.append(ok.astype(f32))

    def shift(v, t):
        if t == 0:
            return v
        return jnp.roll(v, -t, axis=0)

    def conv3(v, wref):
        xs = jnp.concatenate(
            [shift(v, di * _IM + dj) * masks[k]
             for k, (di, dj) in enumerate(_OFFS)], axis=1)
        return jnp.dot(xs, wref[...], preferred_element_type=f32)

    def bnorm(v):
        m = jnp.mean(v, axis=0, keepdims=True)
        var = jnp.mean((v - m) * (v - m), axis=0, keepdims=True)
        return (v - m) * jax.lax.rsqrt(var + 1e-5)

    h = jax.nn.relu(conv3(xf[...], wc1) + bc1[...])
    h = jax.nn.relu(jnp.dot(h, wc2[...], preferred_element_type=f32) + bc2[...])

    y = jax.nn.relu(bnorm(conv3(h, wk1q1)))
    y = bnorm(conv3(y, wk2q2))
    hh = jax.nn.relu(jnp.concatenate([h, h], axis=1) + y)
    kfqf = jnp.dot(hh, wk3q3[...], preferred_element_type=f32) + bkq3[...]
    kf = kfqf[:, :64]
    qf = kfqf[:, 64:]

    qn = qf / (jnp.sqrt(jnp.sum(qf * qf, axis=-1, keepdims=True)) + 1e-8)
    kn = kf / (jnp.sqrt(jnp.sum(kf * kf, axis=-1, keepdims=True)) + 1e-8)

    wr = {}
    for k, (di, dj) in enumerate(_OFFS):
        t = di * _IM + dj
        ks = shift(kn, t) * masks[k]
        w = jnp.sum(qn * ks, axis=-1, keepdims=True)
        wr[(di, dj)] = jnp.roll(w, di * _IM, axis=0) if di else w

    def body(_, s):
        v = {dj: shift(s, dj) for dj in (-1, 0, 1)}
        acc = None
        for di in (-1, 0, 1):
            inner = None
            for dj in (-1, 0, 1):
                term = wr[(di, dj)] * v[dj]
                inner = term if inner is None else inner + term
            inner = shift(inner, di * _IM)
            acc = inner if acc is None else acc + inner
        nrm = jnp.sqrt(jnp.sum(acc * acc, axis=-1, keepdims=True))
        return acc / (nrm + 1e-8)

    s = jax.lax.fori_loop(0, _ITERS, body, s0[...])

    rm = jax.lax.broadcasted_iota(jnp.int32, (_M, _N), 0)
    cm = jax.lax.broadcasted_iota(jnp.int32, (_M, _N), 1)
    sel = (cm == rm * 273).astype(f32)
    agents = jnp.dot(sel, s, preferred_element_type=f32)
    logits_t = jax.lax.dot_general(agents, s, (((1,), (1,)), ((), ())),
                                   preferred_element_type=f32)
    mx = jnp.max(logits_t, axis=0, keepdims=True)
    e = jnp.exp(logits_t - mx)
    out_ref[...] = e / jnp.sum(e, axis=0, keepdims=True)


def _tap_w(w):
    return jnp.transpose(w, (2, 3, 1, 0)).reshape(9 * w.shape[1], w.shape[0])


def _blkdiag2(a, b):
    z = jnp.zeros_like(a)
    return jnp.concatenate(
        [jnp.concatenate([a, z], axis=1), jnp.concatenate([z, b], axis=1)],
        axis=0)


@jax.jit
def kernel(x, Wc1, bc1, Wc2, bc2, Wk1, Wk2, Wk3, bk3, Wq1, Wq2, Wq3,
           init_state, row, col):
    del row, col
    xf = x.reshape(_N, 3)
    wk1q1 = jnp.concatenate([_tap_w(Wk1), _tap_w(Wq1)], axis=1)
    tk2, tq2 = _tap_w(Wk2), _tap_w(Wq2)
    wk2q2 = jnp.concatenate(
        [_blkdiag2(tk2[t * 64:(t + 1) * 64], tq2[t * 64:(t + 1) * 64])
         for t in range(9)], axis=0)
    wk3q3 = _blkdiag2(Wk3[:, :, 0, 0].T, Wq3[:, :, 0, 0].T)
    bkq3 = jnp.concatenate([bk3, jnp.zeros_like(bk3)]).reshape(1, -1)
    args = (
        xf,
        _tap_w(Wc1), bc1.reshape(1, -1),
        Wc2[:, :, 0, 0].T, bc2.reshape(1, -1),
        wk1q1, wk2q2, wk3q3, bkq3,
        init_state.reshape(_N, _Q),
    )
    out = pl.pallas_call(
        _scene_kernel,
        out_shape=jax.ShapeDtypeStruct((_M, _N), jnp.float32),
    )(*args)
    return out.reshape(1, _M, _IM, _IM)


# speedup vs baseline: 1.0242x; 1.0242x over previous
import functools

import jax
import jax.numpy as jnp
from jax.experimental import pallas as pl

_IM = 64
_N = _IM * _IM
_Q = 128
_M = 16
_ITERS = 32
_OFFS = tuple((di, dj) for di in (-1, 0, 1) for dj in (-1, 0, 1))


def _scene_kernel(xf, wc1, bc1, wc2, bc2, wk1, wk2, wk3, bk3, wq1, wq2, wq3,
                  s0, out_ref):
    f32 = jnp.float32
    p = jax.lax.broadcasted_iota(jnp.int32, (_N, 1), 0)
    i = p // _IM
    j = p - i * _IM
    masks = []
    for (di, dj) in _OFFS:
        ii = i + di
        jj = j + dj
        ok = (ii >= 0) & (ii < _IM) & (jj >= 0) & (jj < _IM)
        masks.append(ok.astype(f32))

    def shift(v, t):
        if t == 0:
            return v
        return jnp.roll(v, -t, axis=0)

    def conv3(v, wref, cin):
        acc = None
        for k, (di, dj) in enumerate(_OFFS):
            t = di * _IM + dj
            xs = shift(v, t) * masks[k]
            term = jnp.dot(xs, wref[k * cin:(k + 1) * cin, :],
                           preferred_element_type=f32)
            acc = term if acc is None else acc + term
        return acc

    def bnorm(v):
        m = jnp.mean(v, axis=0, keepdims=True)
        var = jnp.mean((v - m) * (v - m), axis=0, keepdims=True)
        return (v - m) * jax.lax.rsqrt(var + 1e-5)

    def resblock(v, w1, w2):
        y = jax.nn.relu(bnorm(conv3(v, w1, 64)))
        y = bnorm(conv3(y, w2, 64))
        return jax.nn.relu(v + y)

    h = jax.nn.relu(conv3(xf[...], wc1, 3) + bc1[...])
    h = jax.nn.relu(jnp.dot(h, wc2[...], preferred_element_type=f32) + bc2[...])

    kf = jnp.dot(resblock(h, wk1, wk2), wk3[...],
                 preferred_element_type=f32) + bk3[...]
    qf = jnp.dot(resblock(h, wq1, wq2), wq3[...], preferred_element_type=f32)

    qn = qf / (jnp.sqrt(jnp.sum(qf * qf, axis=-1, keepdims=True)) + 1e-8)
    kn = kf / (jnp.sqrt(jnp.sum(kf * kf, axis=-1, keepdims=True)) + 1e-8)

    wr = {}
    for k, (di, dj) in enumerate(_OFFS):
        t = di * _IM + dj
        ks = shift(kn, t) * masks[k]
        w = jnp.sum(qn * ks, axis=-1, keepdims=True)
        wr[(di, dj)] = jnp.roll(w, di * _IM, axis=0) if di else w

    def body(_, s):
        v = {dj: shift(s, dj) for dj in (-1, 0, 1)}
        acc = None
        for di in (-1, 0, 1):
            inner = None
            for dj in (-1, 0, 1):
                term = wr[(di, dj)] * v[dj]
                inner = term if inner is None else inner + term
            inner = shift(inner, di * _IM)
            acc = inner if acc is None else acc + inner
        nrm = jnp.sqrt(jnp.sum(acc * acc, axis=-1, keepdims=True))
        return acc / (nrm + 1e-8)

    s = jax.lax.fori_loop(0, _ITERS, body, s0[...])

    rm = jax.lax.broadcasted_iota(jnp.int32, (_M, _N), 0)
    cm = jax.lax.broadcasted_iota(jnp.int32, (_M, _N), 1)
    sel = (cm == rm * 273).astype(f32)
    agents = jnp.dot(sel, s, preferred_element_type=f32)
    logits_t = jax.lax.dot_general(agents, s, (((1,), (1,)), ((), ())),
                                   preferred_element_type=f32)
    mx = jnp.max(logits_t, axis=0, keepdims=True)
    e = jnp.exp(logits_t - mx)
    out_ref[...] = e / jnp.sum(e, axis=0, keepdims=True)


def _tap_w(w):
    return jnp.transpose(w, (2, 3, 1, 0)).reshape(9 * w.shape[1], w.shape[0])


@jax.jit
def kernel(x, Wc1, bc1, Wc2, bc2, Wk1, Wk2, Wk3, bk3, Wq1, Wq2, Wq3,
           init_state, row, col):
    del row, col
    xf = x.reshape(_N, 3)
    args = (
        xf,
        _tap_w(Wc1), bc1.reshape(1, -1),
        Wc2[:, :, 0, 0].T, bc2.reshape(1, -1),
        _tap_w(Wk1), _tap_w(Wk2), Wk3[:, :, 0, 0].T, bk3.reshape(1, -1),
        _tap_w(Wq1), _tap_w(Wq2), Wq3[:, :, 0, 0].T,
        init_state.reshape(_N, _Q),
    )
    out = pl.pallas_call(
        _scene_kernel,
        out_shape=jax.ShapeDtypeStruct((_M, _N), jnp.float32),
    )(*args)
    return out.reshape(1, _M, _IM, _IM)


# speedup vs baseline: 1.9582x; 1.9119x over previous
import jax
import jax.numpy as jnp
from jax.experimental import pallas as pl
from jax.experimental.pallas import tpu as pltpu

_IM = 64
_N = _IM * _IM
_Q = 128
_M = 16
_ITERS = 32
_PAD = _IM + 1
_OFFS = tuple((di, dj) for di in (-1, 0, 1) for dj in (-1, 0, 1))


def _scene_kernel(xf, wc1, bc1, wc2, bc2, wk1, wk2, wk3, bk3, wq1, wq2, wq3,
                  s0, out_ref, spad):
    f32 = jnp.float32
    p = jax.lax.broadcasted_iota(jnp.int32, (_N, 1), 0)
    i = p // _IM
    j = p - i * _IM
    masks = []
    for (di, dj) in _OFFS:
        ii = i + di
        jj = j + dj
        ok = (ii >= 0) & (ii < _IM) & (jj >= 0) & (jj < _IM)
        masks.append(ok.astype(f32))

    def shift(v, t):
        if t == 0:
            return v
        return jnp.roll(v, -t, axis=0)

    def conv3(v, wref, cin):
        acc = None
        for k, (di, dj) in enumerate(_OFFS):
            t = di * _IM + dj
            xs = shift(v, t) * masks[k]
            term = jnp.dot(xs, wref[k * cin:(k + 1) * cin, :],
                           preferred_element_type=f32)
            acc = term if acc is None else acc + term
        return acc

    def bnorm(v):
        m = jnp.mean(v, axis=0, keepdims=True)
        var = jnp.mean((v - m) * (v - m), axis=0, keepdims=True)
        return (v - m) * jax.lax.rsqrt(var + 1e-5)

    def resblock(v, w1, w2):
        y = jax.nn.relu(bnorm(conv3(v, w1, 64)))
        y = bnorm(conv3(y, w2, 64))
        return jax.nn.relu(v + y)

    h = jax.nn.relu(conv3(xf[...], wc1, 3) + bc1[...])
    h = jax.nn.relu(jnp.dot(h, wc2[...], preferred_element_type=f32) + bc2[...])

    kf = jnp.dot(resblock(h, wk1, wk2), wk3[...],
                 preferred_element_type=f32) + bk3[...]
    qf = jnp.dot(resblock(h, wq1, wq2), wq3[...], preferred_element_type=f32)

    qn = qf / (jnp.sqrt(jnp.sum(qf * qf, axis=-1, keepdims=True)) + 1e-8)
    kn = kf / (jnp.sqrt(jnp.sum(kf * kf, axis=-1, keepdims=True)) + 1e-8)

    wd = []
    for k, (di, dj) in enumerate(_OFFS):
        t = di * _IM + dj
        ks = shift(kn, t) * masks[k]
        wd.append(jnp.sum(qn * ks, axis=-1, keepdims=True))

    spad[pl.ds(0, _PAD), :] = jnp.zeros((_PAD, _Q), f32)
    spad[pl.ds(_PAD + _N, _PAD), :] = jnp.zeros((_PAD, _Q), f32)
    spad[pl.ds(_PAD, _N), :] = s0[...]

    def body(_, carry):
        acc = None
        for k, (di, dj) in enumerate(_OFFS):
            t = di * _IM + dj
            term = wd[k] * spad[pl.ds(_PAD + t, _N), :]
            acc = term if acc is None else acc + term
        inv = 1.0 / (jnp.sqrt(jnp.sum(acc * acc, axis=-1, keepdims=True))
                     + 1e-8)
        spad[pl.ds(_PAD, _N), :] = acc * inv
        return carry

    jax.lax.fori_loop(0, _ITERS, body, 0)
    s = spad[pl.ds(_PAD, _N), :]

    rm = jax.lax.broadcasted_iota(jnp.int32, (_M, _N), 0)
    cm = jax.lax.broadcasted_iota(jnp.int32, (_M, _N), 1)
    sel = (cm == rm * 273).astype(f32)
    agents = jnp.dot(sel, s, preferred_element_type=f32)
    logits_t = jax.lax.dot_general(agents, s, (((1,), (1,)), ((), ())),
                                   preferred_element_type=f32)
    mx = jnp.max(logits_t, axis=0, keepdims=True)
    e = jnp.exp(logits_t - mx)
    out_ref[...] = e / jnp.sum(e, axis=0, keepdims=True)


def _tap_w(w):
    return jnp.transpose(w, (2, 3, 1, 0)).reshape(9 * w.shape[1], w.shape[0])


@jax.jit
def kernel(x, Wc1, bc1, Wc2, bc2, Wk1, Wk2, Wk3, bk3, Wq1, Wq2, Wq3,
           init_state, row, col):
    del row, col
    xf = x.reshape(_N, 3)
    args = (
        xf,
        _tap_w(Wc1), bc1.reshape(1, -1),
        Wc2[:, :, 0, 0].T, bc2.reshape(1, -1),
        _tap_w(Wk1), _tap_w(Wk2), Wk3[:, :, 0, 0].T, bk3.reshape(1, -1),
        _tap_w(Wq1), _tap_w(Wq2), Wq3[:, :, 0, 0].T,
        init_state.reshape(_N, _Q),
    )
    out = pl.pallas_call(
        _scene_kernel,
        out_shape=jax.ShapeDtypeStruct((_M, _N), jnp.float32),
        scratch_shapes=[pltpu.VMEM((2 * _PAD + _N, _Q), jnp.float32)],
    )(*args)
    return out.reshape(1, _M, _IM, _IM)


# speedup vs baseline: 2.0500x; 1.0469x over previous
import jax
import jax.numpy as jnp
from jax.experimental import pallas as pl
from jax.experimental.pallas import tpu as pltpu

_IM = 64
_N = _IM * _IM
_Q = 128
_M = 16
_ITERS = 32
_PAD = _IM + 1
_OFFS = tuple((di, dj) for di in (-1, 0, 1) for dj in (-1, 0, 1))


def _scene_kernel(xf, wc1, bc1, wc2, bc2, wk1, wk2, wk3, bk3, wq1, wq2, wq3,
                  s0, out_ref, spad, xpad):
    f32 = jnp.float32
    p = jax.lax.broadcasted_iota(jnp.int32, (_N, 1), 0)
    i = p // _IM
    j = p - i * _IM
    masks = []
    for (di, dj) in _OFFS:
        ii = i + di
        jj = j + dj
        ok = (ii >= 0) & (ii < _IM) & (jj >= 0) & (jj < _IM)
        masks.append(ok.astype(f32))

    def shift(v, t):
        if t == 0:
            return v
        return jnp.roll(v, -t, axis=0)

    zpad = jnp.zeros((_PAD, 64), f32)
    xpad[pl.ds(0, _PAD), :] = zpad
    xpad[pl.ds(_PAD + _N, _PAD), :] = zpad

    jmask = {dj: ((j + dj >= 0) & (j + dj < _IM)).astype(f32)
             for dj in (-1, 1)}

    def conv3(v, wref):
        xpad[pl.ds(_PAD, _N), :] = v
        acc = None
        for dj in (-1, 0, 1):
            sub = None
            for di in (-1, 0, 1):
                k = (di + 1) * 3 + (dj + 1)
                t = di * _IM + dj
                term = jnp.dot(xpad[pl.ds(_PAD + t, _N), :],
                               wref[k * 64:(k + 1) * 64, :],
                               preferred_element_type=f32)
                sub = term if sub is None else sub + term
            if dj != 0:
                sub = sub * jmask[dj]
            acc = sub if acc is None else acc + sub
        return acc

    def conv3_first(v, wref, cin):
        acc = None
        for k, (di, dj) in enumerate(_OFFS):
            t = di * _IM + dj
            xs = shift(v, t) * masks[k]
            term = jnp.dot(xs, wref[k * cin:(k + 1) * cin, :],
                           preferred_element_type=f32)
            acc = term if acc is None else acc + term
        return acc

    def bnorm(v):
        m = jnp.mean(v, axis=0, keepdims=True)
        var = jnp.mean((v - m) * (v - m), axis=0, keepdims=True)
        return (v - m) * jax.lax.rsqrt(var + 1e-5)

    def resblock(v, w1, w2):
        y = jax.nn.relu(bnorm(conv3(v, w1)))
        y = bnorm(conv3(y, w2))
        return jax.nn.relu(v + y)

    h = jax.nn.relu(conv3_first(xf[...], wc1, 3) + bc1[...])
    h = jax.nn.relu(jnp.dot(h, wc2[...], preferred_element_type=f32) + bc2[...])

    kf = jnp.dot(resblock(h, wk1, wk2), wk3[...],
                 preferred_element_type=f32) + bk3[...]
    qf = jnp.dot(resblock(h, wq1, wq2), wq3[...], preferred_element_type=f32)

    qn = qf / (jnp.sqrt(jnp.sum(qf * qf, axis=-1, keepdims=True)) + 1e-8)
    kn = kf / (jnp.sqrt(jnp.sum(kf * kf, axis=-1, keepdims=True)) + 1e-8)

    xpad[pl.ds(_PAD, _N), :] = kn
    wd = []
    for k, (di, dj) in enumerate(_OFFS):
        t = di * _IM + dj
        w = jnp.sum(qn * xpad[pl.ds(_PAD + t, _N), :], axis=-1, keepdims=True)
        if dj != 0:
            w = w * jmask[dj]
        wd.append(w)

    zq = jnp.zeros((_PAD, _Q), f32)
    spad[pl.ds(0, _PAD), :] = zq
    spad[pl.ds(_PAD + _N, _PAD), :] = zq
    spad[pl.ds(_PAD, _N), :] = s0[...]

    def body(_, carry):
        acc = None
        for k, (di, dj) in enumerate(_OFFS):
            t = di * _IM + dj
            term = wd[k] * spad[pl.ds(_PAD + t, _N), :]
            acc = term if acc is None else acc + term
        inv = 1.0 / (jnp.sqrt(jnp.sum(acc * acc, axis=-1, keepdims=True))
                     + 1e-8)
        spad[pl.ds(_PAD, _N), :] = acc * inv
        return carry

    jax.lax.fori_loop(0, _ITERS, body, 0)
    s = spad[pl.ds(_PAD, _N), :]

    rm = jax.lax.broadcasted_iota(jnp.int32, (_M, _N), 0)
    cm = jax.lax.broadcasted_iota(jnp.int32, (_M, _N), 1)
    sel = (cm == rm * 273).astype(f32)
    agents = jnp.dot(sel, s, preferred_element_type=f32)
    logits_t = jax.lax.dot_general(agents, s, (((1,), (1,)), ((), ())),
                                   preferred_element_type=f32)
    mx = jnp.max(logits_t, axis=0, keepdims=True)
    e = jnp.exp(logits_t - mx)
    out_ref[...] = e / jnp.sum(e, axis=0, keepdims=True)


def _tap_w(w):
    return jnp.transpose(w, (2, 3, 1, 0)).reshape(9 * w.shape[1], w.shape[0])


@jax.jit
def kernel(x, Wc1, bc1, Wc2, bc2, Wk1, Wk2, Wk3, bk3, Wq1, Wq2, Wq3,
           init_state, row, col):
    del row, col
    xf = x.reshape(_N, 3)
    args = (
        xf,
        _tap_w(Wc1), bc1.reshape(1, -1),
        Wc2[:, :, 0, 0].T, bc2.reshape(1, -1),
        _tap_w(Wk1), _tap_w(Wk2), Wk3[:, :, 0, 0].T, bk3.reshape(1, -1),
        _tap_w(Wq1), _tap_w(Wq2), Wq3[:, :, 0, 0].T,
        init_state.reshape(_N, _Q),
    )
    out = pl.pallas_call(
        _scene_kernel,
        out_shape=jax.ShapeDtypeStruct((_M, _N), jnp.float32),
        scratch_shapes=[pltpu.VMEM((2 * _PAD + _N, _Q), jnp.float32),
                        pltpu.VMEM((2 * _PAD + _N, 64), jnp.float32)],
    )(*args)
    return out.reshape(1, _M, _IM, _IM)
